# baseline (speedup 1.0000x reference)
import functools

import jax
import jax.numpy as jnp
from jax import lax
from jax.experimental import pallas as pl
from jax.experimental.pallas import tpu as pltpu
from jax.experimental.pallas import tpu_sc as plsc

E = 8
D = 768
F = 768
DP = D // 2
T = 2048
BLK = 512
MAXB = 15
NPAD = MAXB * BLK
NC = 2
NS = 16
NW = NC * NS
TPW = T // NW


def _pack(x):
    z = lax.bitcast_convert_type(x.astype(jnp.bfloat16), jnp.uint16)
    lo = z[:, :DP].astype(jnp.uint32)
    hi = z[:, DP:].astype(jnp.uint32)
    return lax.bitcast_convert_type(lo | (hi << 16), jnp.int32)


def _unpack(p):
    u = lax.bitcast_convert_type(p, jnp.uint32)
    lo = lax.bitcast_convert_type((u & 0xFFFF).astype(jnp.uint16), jnp.bfloat16)
    hi = lax.bitcast_convert_type((u >> 16).astype(jnp.uint16), jnp.bfloat16)
    return jnp.concatenate([lo, hi], axis=1)


NTB = 4
TB = T // NTB


def _router_body(h2_ref, gw_ref, ht_ref, pa_ref, pb_ref, wt_ref, be_ref, bv_ref,
                 ra_s, rb_s, ea_s, eb_s, carry_s, poff_s):
    p = pl.program_id(0)
    t = pl.program_id(1)
    iota_e = lax.broadcasted_iota(jnp.int32, (TB, E), 1)

    @pl.when(p == 0)
    def _phase0():
        h = h2_ref[...]
        gw = gw_ref[...]
        ht_ref[...] = _pack(jnp.transpose(h, (1, 0)))
        logits = lax.dot_general(h, gw, (((0,), (1,)), ((), ())),
                                 preferred_element_type=jnp.float32)
        m1 = jnp.max(logits, axis=1, keepdims=True)
        e1 = jnp.min(jnp.where(logits >= m1, iota_e, E), axis=1, keepdims=True)
        a_mask = iota_e == e1
        l2 = jnp.where(a_mask, -jnp.inf, logits)
        m2 = jnp.max(l2, axis=1, keepdims=True)
        e2 = jnp.min(jnp.where(l2 >= m2, iota_e, E), axis=1, keepdims=True)
        b_mask = iota_e == e2
        w1 = 1.0 / (1.0 + jnp.exp(m2 - m1))
        wt_ref[:, 0:1] = w1
        wt_ref[:, 1:2] = 1.0 - w1

        s = a_mask.astype(jnp.float32) + b_mask.astype(jnp.float32)
        x = s
        shift = 1
        while shift < TB:
            x = x + jnp.concatenate(
                [jnp.zeros((shift, E), jnp.float32), x[: TB - shift]], axis=0)
            shift *= 2
        carry = jnp.where(t == 0, jnp.zeros((1, E), jnp.float32), carry_s[...])
        rank = x - s + carry
        carry_s[...] = carry + x[TB - 1:TB, :]
        ra_s[0, pl.ds(t * TB, TB)] = jnp.sum(jnp.where(a_mask, rank, 0.0), axis=1)
        rb_s[0, pl.ds(t * TB, TB)] = jnp.sum(jnp.where(b_mask, rank, 0.0), axis=1)
        ea_s[0, pl.ds(t * TB, TB)] = e1[:, 0]
        eb_s[0, pl.ds(t * TB, TB)] = e2[:, 0]

    @pl.when(p == 1)
    def _phase1():
        @pl.when(t == 0)
        def _():
            counts = carry_s[...]
            nblk = jnp.ceil(counts * (1.0 / BLK))
            tri = (lax.broadcasted_iota(jnp.int32, (E, E), 0)
                   <= lax.broadcasted_iota(jnp.int32, (E, E), 1)
                   ).astype(jnp.float32)
            cumb = lax.dot_general(nblk, tri, (((1,), (0,)), ((), ())),
                                   preferred_element_type=jnp.float32)
            poff_s[...] = (cumb - nblk) * BLK
            ib = lax.broadcasted_iota(jnp.int32, (MAXB, E), 0)
            cumb_i = cumb.astype(jnp.int32)
            owner = jnp.sum((ib >= cumb_i).astype(jnp.int32), axis=1)
            bv_ref[0, :] = (owner < E).astype(jnp.int32)
            be_ref[0, :] = jnp.minimum(owner, E - 1)

        poff = poff_s[...]
        ea = ea_s[0, pl.ds(t * TB, TB)]
        eb = eb_s[0, pl.ds(t * TB, TB)]
        off_a = jnp.sum(jnp.where(ea[:, None] == iota_e, poff, 0.0), axis=1)
        off_b = jnp.sum(jnp.where(eb[:, None] == iota_e, poff, 0.0), axis=1)
        pa_ref[0, :] = (ra_s[0, pl.ds(t * TB, TB)] + off_a).astype(jnp.int32)
        pb_ref[0, :] = (rb_s[0, pl.ds(t * TB, TB)] + off_b).astype(jnp.int32)


def _router(h2, gate_w):
    return pl.pallas_call(
        _router_body,
        grid=(2, NTB),
        in_specs=[
            pl.BlockSpec((D, TB), lambda p, t: (0, jnp.where(p == 0, t, NTB - 1))),
            pl.BlockSpec((E, D), lambda p, t: (0, 0)),
        ],
        out_specs=[
            pl.BlockSpec((TB, DP), lambda p, t: (jnp.where(p == 0, t, NTB - 1), 0)),
            pl.BlockSpec((1, TB), lambda p, t: (0, jnp.where(p == 1, t, 0))),
            pl.BlockSpec((1, TB), lambda p, t: (0, jnp.where(p == 1, t, 0))),
            pl.BlockSpec((TB, 2), lambda p, t: (jnp.where(p == 0, t, NTB - 1), 0)),
            pl.BlockSpec((1, MAXB), lambda p, t: (0, 0)),
            pl.BlockSpec((1, MAXB), lambda p, t: (0, 0)),
        ],
        out_shape=[
            jax.ShapeDtypeStruct((T, DP), jnp.int32),
            jax.ShapeDtypeStruct((1, T), jnp.int32),
            jax.ShapeDtypeStruct((1, T), jnp.int32),
            jax.ShapeDtypeStruct((T, 2), jnp.float32),
            jax.ShapeDtypeStruct((1, MAXB), jnp.int32),
            jax.ShapeDtypeStruct((1, MAXB), jnp.int32),
        ],
        scratch_shapes=[
            pltpu.VMEM((1, T), jnp.float32),
            pltpu.VMEM((1, T), jnp.float32),
            pltpu.VMEM((1, T), jnp.int32),
            pltpu.VMEM((1, T), jnp.int32),
            pltpu.VMEM((1, E), jnp.float32),
            pltpu.VMEM((1, E), jnp.float32),
        ],
    )(h2, gate_w)


@functools.cache
def _make_sc_scatter():
    @functools.partial(
        pl.kernel,
        mesh=plsc.VectorSubcoreMesh(core_axis_name="c", subcore_axis_name="s"),
        out_type=jax.ShapeDtypeStruct((NPAD, DP), jnp.int32),
        scratch_types=[
            pltpu.VMEM((TPW,), jnp.int32),
            pltpu.VMEM((TPW,), jnp.int32),
            pltpu.VMEM((TPW, DP), jnp.int32),
            pltpu.SemaphoreType.DMA,
        ],
    )
    def _sc_scatter(h_hbm, pos_a_hbm, pos_b_hbm, out_hbm, ia_v, ib_v, rows_v, sem):
        wid = lax.axis_index("s") * NC + lax.axis_index("c")
        start = wid * TPW
        pltpu.sync_copy(h_hbm.at[pl.ds(start, TPW)], rows_v)
        pltpu.sync_copy(pos_a_hbm.at[pl.ds(start, TPW)], ia_v)
        pltpu.sync_copy(pos_b_hbm.at[pl.ds(start, TPW)], ib_v)
        pltpu.async_copy(rows_v, out_hbm.at[ia_v], sem).wait()
        pltpu.async_copy(rows_v, out_hbm.at[ib_v], sem).wait()

    return _sc_scatter


def _expert_body(be_ref, bv_ref, x_ref, w1_ref, w3_ref, w2_ref, y_ref):
    i = pl.program_id(0)

    @pl.when(bv_ref[i] != 0)
    def _():
        x = _unpack(x_ref[...])
        w1 = w1_ref[0].astype(jnp.bfloat16)
        w3 = w3_ref[0].astype(jnp.bfloat16)
        w2 = w2_ref[0].astype(jnp.bfloat16)
        a = lax.dot_general(x, w1, (((1,), (1,)), ((), ())),
                            preferred_element_type=jnp.float32)
        b = lax.dot_general(x, w3, (((1,), (1,)), ((), ())),
                            preferred_element_type=jnp.float32)
        g = 0.5 * a * (1.0 + lax.erf(a * 0.7071067811865476))
        inter = (g * b).astype(jnp.bfloat16)
        y = lax.dot_general(inter, w2, (((1,), (1,)), ((), ())),
                            preferred_element_type=jnp.float32)
        y_ref[...] = _pack(y)


def _experts(hg, W1, W2, W3, be, bv):
    grid_spec = pltpu.PrefetchScalarGridSpec(
        num_scalar_prefetch=2,
        grid=(MAXB,),
        in_specs=[
            pl.BlockSpec((BLK, DP), lambda i, be, bv: (i, 0)),
            pl.BlockSpec((1, F, D), lambda i, be, bv: (be[i], 0, 0)),
            pl.BlockSpec((1, F, D), lambda i, be, bv: (be[i], 0, 0)),
            pl.BlockSpec((1, D, F), lambda i, be, bv: (be[i], 0, 0)),
        ],
        out_specs=pl.BlockSpec((BLK, DP), lambda i, be, bv: (i, 0)),
    )
    return pl.pallas_call(
        _expert_body,
        grid_spec=grid_spec,
        out_shape=jax.ShapeDtypeStruct((NPAD, DP), jnp.int32),
    )(be, bv, hg, W1, W3, W2)


@functools.cache
def _make_sc_gather():
    @functools.partial(
        pl.kernel,
        mesh=plsc.VectorSubcoreMesh(core_axis_name="c", subcore_axis_name="s"),
        out_type=[
            jax.ShapeDtypeStruct((T, DP), jnp.int32),
            jax.ShapeDtypeStruct((T, DP), jnp.int32),
        ],
        scratch_types=[
            pltpu.VMEM((TPW,), jnp.int32),
            pltpu.VMEM((TPW, DP), jnp.int32),
            pltpu.SemaphoreType.DMA,
        ],
    )
    def _sc_gather(yg_hbm, pos_a_hbm, pos_b_hbm, oa_hbm, ob_hbm, idx_v, rows_v, sem):
        wid = lax.axis_index("s") * NC + lax.axis_index("c")
        start = wid * TPW
        pltpu.sync_copy(pos_a_hbm.at[pl.ds(start, TPW)], idx_v)
        pltpu.async_copy(yg_hbm.at[idx_v], rows_v, sem).wait()
        pltpu.sync_copy(rows_v, oa_hbm.at[pl.ds(start, TPW)])
        pltpu.sync_copy(pos_b_hbm.at[pl.ds(start, TPW)], idx_v)
        pltpu.async_copy(yg_hbm.at[idx_v], rows_v, sem).wait()
        pltpu.sync_copy(rows_v, ob_hbm.at[pl.ds(start, TPW)])

    return _sc_gather


def _combine_body(a_ref, b_ref, wt_ref, o_ref):
    ya = _unpack(a_ref[...]).astype(jnp.float32)
    yb = _unpack(b_ref[...]).astype(jnp.float32)
    o_ref[...] = jnp.transpose(
        wt_ref[:, 0:1] * ya + wt_ref[:, 1:2] * yb, (1, 0))


def _combine(ya, yb, wt):
    blk = 512
    return pl.pallas_call(
        _combine_body,
        grid=(T // blk,),
        in_specs=[
            pl.BlockSpec((blk, DP), lambda i: (i, 0)),
            pl.BlockSpec((blk, DP), lambda i: (i, 0)),
            pl.BlockSpec((blk, 2), lambda i: (i, 0)),
        ],
        out_specs=pl.BlockSpec((D, blk), lambda i: (0, i)),
        out_shape=jax.ShapeDtypeStruct((D, T), jnp.float32),
    )(ya, yb, wt)


def kernel(hidden_states, gate_w, W1, W2, W3):
    ht, pa, pb, wt, be, bv = _router(hidden_states[0], gate_w)
    pos_a = pa.reshape(T)
    pos_b = pb.reshape(T)
    hg = _make_sc_scatter()(ht, pos_a, pos_b)
    yg = _experts(hg, W1, W2, W3, be.reshape(MAXB), bv.reshape(MAXB))
    ya, yb = _make_sc_gather()(yg, pos_a, pos_b)
    return _combine(ya, yb, wt)[None]


# speedup vs baseline: 1.0643x; 1.0643x over previous
import functools

import jax
import jax.numpy as jnp
from jax import lax
from jax.experimental import pallas as pl
from jax.experimental.pallas import tpu as pltpu
from jax.experimental.pallas import tpu_sc as plsc

E = 8
D = 768
F = 768
DP = D // 2
T = 2048
BLK = 512
MAXB = 15
NPAD = MAXB * BLK
NC = 2
NS = 16
NW = NC * NS
TPW = T // NW


def _pack(x):
    z = lax.bitcast_convert_type(x.astype(jnp.bfloat16), jnp.uint16)
    lo = z[:, :DP].astype(jnp.uint32)
    hi = z[:, DP:].astype(jnp.uint32)
    return lax.bitcast_convert_type(lo | (hi << 16), jnp.int32)


def _unpack(p):
    u = lax.bitcast_convert_type(p, jnp.uint32)
    lo = lax.bitcast_convert_type((u & 0xFFFF).astype(jnp.uint16), jnp.bfloat16)
    hi = lax.bitcast_convert_type((u >> 16).astype(jnp.uint16), jnp.bfloat16)
    return jnp.concatenate([lo, hi], axis=1)


def _router_body(h2_ref, gw_ref, ht_ref, pa_ref, pb_ref, wt_ref, be_ref, bv_ref):
    h2 = h2_ref[...]
    gw = gw_ref[...]
    ht_ref[...] = _pack(jnp.transpose(h2, (1, 0)))
    logits = lax.dot_general(h2, gw, (((0,), (1,)), ((), ())),
                             preferred_element_type=jnp.float32)
    iota_e = lax.broadcasted_iota(jnp.int32, (T, E), 1)
    m1 = jnp.max(logits, axis=1, keepdims=True)
    e1 = jnp.min(jnp.where(logits >= m1, iota_e, E), axis=1, keepdims=True)
    a_mask = iota_e == e1
    l2 = jnp.where(a_mask, -jnp.inf, logits)
    m2 = jnp.max(l2, axis=1, keepdims=True)
    e2 = jnp.min(jnp.where(l2 >= m2, iota_e, E), axis=1, keepdims=True)
    b_mask = iota_e == e2
    w1 = 1.0 / (1.0 + jnp.exp(m2 - m1))
    wt_ref[:, 0:1] = w1
    wt_ref[:, 1:2] = 1.0 - w1

    s = a_mask.astype(jnp.float32) + b_mask.astype(jnp.float32)
    x = s
    shift = 1
    while shift < T:
        x = x + jnp.concatenate(
            [jnp.zeros((shift, E), jnp.float32), x[: T - shift]], axis=0)
        shift *= 2
    rank = x - s
    counts = jnp.sum(s, axis=0, keepdims=True)
    nblk = jnp.ceil(counts * (1.0 / BLK))
    tri = (lax.broadcasted_iota(jnp.int32, (E, E), 0)
           <= lax.broadcasted_iota(jnp.int32, (E, E), 1)).astype(jnp.float32)
    cumb = lax.dot_general(nblk, tri, (((1,), (0,)), ((), ())),
                           preferred_element_type=jnp.float32)
    poff = (cumb - nblk) * BLK
    base = poff + rank
    pos_a = jnp.sum(jnp.where(a_mask, base, 0.0), axis=1)
    pos_b = jnp.sum(jnp.where(b_mask, base, 0.0), axis=1)
    pa_ref[0, :] = pos_a.astype(jnp.int32)
    pb_ref[0, :] = pos_b.astype(jnp.int32)

    ib = lax.broadcasted_iota(jnp.int32, (MAXB, E), 0)
    cumb_i = cumb.astype(jnp.int32)
    owner = jnp.sum((ib >= cumb_i).astype(jnp.int32), axis=1)
    bv_ref[0, :] = (owner < E).astype(jnp.int32)
    be_ref[0, :] = jnp.minimum(owner, E - 1)


def _router(h2, gate_w):
    return pl.pallas_call(
        _router_body,
        out_shape=[
            jax.ShapeDtypeStruct((T, DP), jnp.int32),
            jax.ShapeDtypeStruct((1, T), jnp.int32),
            jax.ShapeDtypeStruct((1, T), jnp.int32),
            jax.ShapeDtypeStruct((T, 2), jnp.float32),
            jax.ShapeDtypeStruct((1, MAXB), jnp.int32),
            jax.ShapeDtypeStruct((1, MAXB), jnp.int32),
        ],
    )(h2, gate_w)


@functools.cache
def _make_sc_scatter():
    @functools.partial(
        pl.kernel,
        mesh=plsc.VectorSubcoreMesh(core_axis_name="c", subcore_axis_name="s"),
        out_type=jax.ShapeDtypeStruct((NPAD, DP), jnp.int32),
        scratch_types=[
            pltpu.VMEM((TPW,), jnp.int32),
            pltpu.VMEM((TPW,), jnp.int32),
            pltpu.VMEM((TPW, DP), jnp.int32),
            pltpu.SemaphoreType.DMA,
        ],
    )
    def _sc_scatter(h_hbm, pos_a_hbm, pos_b_hbm, out_hbm, ia_v, ib_v, rows_v, sem):
        wid = lax.axis_index("s") * NC + lax.axis_index("c")
        start = wid * TPW
        pltpu.sync_copy(h_hbm.at[pl.ds(start, TPW)], rows_v)
        pltpu.sync_copy(pos_a_hbm.at[pl.ds(start, TPW)], ia_v)
        pltpu.sync_copy(pos_b_hbm.at[pl.ds(start, TPW)], ib_v)
        pltpu.async_copy(rows_v, out_hbm.at[ia_v], sem).wait()
        pltpu.async_copy(rows_v, out_hbm.at[ib_v], sem).wait()

    return _sc_scatter


def _expert_body(be_ref, bv_ref, x_ref, w1_ref, w3_ref, w2_ref, y_ref):
    i = pl.program_id(0)

    @pl.when(bv_ref[i] != 0)
    def _():
        x = _unpack(x_ref[...])
        w1 = w1_ref[0].astype(jnp.bfloat16)
        w3 = w3_ref[0].astype(jnp.bfloat16)
        w2 = w2_ref[0].astype(jnp.bfloat16)
        a = lax.dot_general(x, w1, (((1,), (1,)), ((), ())),
                            preferred_element_type=jnp.float32)
        b = lax.dot_general(x, w3, (((1,), (1,)), ((), ())),
                            preferred_element_type=jnp.float32)
        g = 0.5 * a * (1.0 + lax.erf(a * 0.7071067811865476))
        inter = (g * b).astype(jnp.bfloat16)
        y = lax.dot_general(inter, w2, (((1,), (1,)), ((), ())),
                            preferred_element_type=jnp.float32)
        y_ref[...] = _pack(y)


def _experts(hg, W1, W2, W3, be, bv):
    grid_spec = pltpu.PrefetchScalarGridSpec(
        num_scalar_prefetch=2,
        grid=(MAXB,),
        in_specs=[
            pl.BlockSpec((BLK, DP), lambda i, be, bv: (i, 0)),
            pl.BlockSpec((1, F, D), lambda i, be, bv: (be[i], 0, 0)),
            pl.BlockSpec((1, F, D), lambda i, be, bv: (be[i], 0, 0)),
            pl.BlockSpec((1, D, F), lambda i, be, bv: (be[i], 0, 0)),
        ],
        out_specs=pl.BlockSpec((BLK, DP), lambda i, be, bv: (i, 0)),
    )
    return pl.pallas_call(
        _expert_body,
        grid_spec=grid_spec,
        out_shape=jax.ShapeDtypeStruct((NPAD, DP), jnp.int32),
    )(be, bv, hg, W1, W3, W2)


@functools.cache
def _make_sc_gather():
    @functools.partial(
        pl.kernel,
        mesh=plsc.VectorSubcoreMesh(core_axis_name="c", subcore_axis_name="s"),
        out_type=[
            jax.ShapeDtypeStruct((T, DP), jnp.int32),
            jax.ShapeDtypeStruct((T, DP), jnp.int32),
        ],
        scratch_types=[
            pltpu.VMEM((TPW,), jnp.int32),
            pltpu.VMEM((TPW, DP), jnp.int32),
            pltpu.SemaphoreType.DMA,
        ],
    )
    def _sc_gather(yg_hbm, pos_a_hbm, pos_b_hbm, oa_hbm, ob_hbm, idx_v, rows_v, sem):
        wid = lax.axis_index("s") * NC + lax.axis_index("c")
        start = wid * TPW
        pltpu.sync_copy(pos_a_hbm.at[pl.ds(start, TPW)], idx_v)
        pltpu.async_copy(yg_hbm.at[idx_v], rows_v, sem).wait()
        pltpu.sync_copy(rows_v, oa_hbm.at[pl.ds(start, TPW)])
        pltpu.sync_copy(pos_b_hbm.at[pl.ds(start, TPW)], idx_v)
        pltpu.async_copy(yg_hbm.at[idx_v], rows_v, sem).wait()
        pltpu.sync_copy(rows_v, ob_hbm.at[pl.ds(start, TPW)])

    return _sc_gather


def _combine_body(a_ref, b_ref, wt_ref, o_ref):
    ya = _unpack(a_ref[...]).astype(jnp.float32)
    yb = _unpack(b_ref[...]).astype(jnp.float32)
    o_ref[...] = jnp.transpose(
        wt_ref[:, 0:1] * ya + wt_ref[:, 1:2] * yb, (1, 0))


def _combine(ya, yb, wt):
    blk = 512
    return pl.pallas_call(
        _combine_body,
        grid=(T // blk,),
        in_specs=[
            pl.BlockSpec((blk, DP), lambda i: (i, 0)),
            pl.BlockSpec((blk, DP), lambda i: (i, 0)),
            pl.BlockSpec((blk, 2), lambda i: (i, 0)),
        ],
        out_specs=pl.BlockSpec((D, blk), lambda i: (0, i)),
        out_shape=jax.ShapeDtypeStruct((D, T), jnp.float32),
    )(ya, yb, wt)


def kernel(hidden_states, gate_w, W1, W2, W3):
    ht, pa, pb, wt, be, bv = _router(hidden_states[0], gate_w)
    pos_a = pa.reshape(T)
    pos_b = pb.reshape(T)
    hg = _make_sc_scatter()(ht, pos_a, pos_b)
    yg = _experts(hg, W1, W2, W3, be.reshape(MAXB), bv.reshape(MAXB))
    ya, yb = _make_sc_gather()(yg, pos_a, pos_b)
    return _combine(ya, yb, wt)[None]
